# Initial kernel scaffold; baseline (speedup 1.0000x reference)
import jax
import jax.numpy as jnp
from jax.experimental import pallas as pl


def kernel(x, edge_index, edge_attr, W1, b1, W2, b2):
    raise NotImplementedError("write your pallas kernel here")


# speedup vs baseline: 2.5574x; 2.5574x over previous
import functools

import jax
import jax.numpy as jnp
from jax import lax
from jax.experimental import pallas as pl
from jax.experimental.pallas import tpu as pltpu
from jax.experimental.pallas import tpu_sc as plsc

N = 10000
E = 320000
D = 128
DH = 144
D_EDGE = 16

NC = 2
NS = 16
NW = NC * NS
EPW = E // NW
B = 40
NCHUNK = EPW // B
NPAD = 10112
NPT = NPAD // NS
WB = NPT // B
WTAIL = NPT - WB * B


def _node_mm_body(x_ref, wa_ref, wb_ref, xa_ref, xb_ref):
    xv = x_ref[...]
    xa_ref[...] = jnp.dot(xv, wa_ref[...], preferred_element_type=jnp.float32)
    xb_ref[...] = jnp.dot(xv, wb_ref[...], preferred_element_type=jnp.float32)


def _edge_mm_body(attr_ref, we_ref, b1_ref, ea_ref):
    ea_ref[...] = (
        jnp.dot(attr_ref[...], we_ref[...], preferred_element_type=jnp.float32)
        + b1_ref[...]
    )


def _cnt_mm_body(col_ref, cnt_ref):
    i = pl.program_id(0)
    colv = col_ref[...]
    hi = lax.shift_right_logical(colv, 7)
    lo = lax.bitwise_and(colv, 127)
    r80 = lax.broadcasted_iota(jnp.int32, (1, 80), 1)
    l128 = lax.broadcasted_iota(jnp.int32, (1, D), 1)
    m = (hi == r80).astype(jnp.float32)
    p = (lo == l128).astype(jnp.float32)
    acc = lax.dot_general(m, p, (((0,), (0,)), ((), ())),
                          preferred_element_type=jnp.float32)

    @pl.when(i == 0)
    def _():
        cnt_ref[...] = jnp.zeros_like(cnt_ref)

    cnt_ref[...] += acc


def _final_body(h0_ref, h1_ref, cnt_ref, w2_ref, b2_ref, out_ref):
    hsum = h0_ref[...] + h1_ref[...]
    cnt = cnt_ref[...]
    acc = jnp.dot(hsum, w2_ref[...], preferred_element_type=jnp.float32)
    out_ref[...] = (acc + cnt * b2_ref[...]) / jnp.maximum(cnt, 1.0)


def _sc_edge_body(row_hbm, col_hbm, ea_hbm, xa_hbm, xb_hbm, zh_hbm,
                  h_out,
                  row_v, col_v, buf_a, buf_b, buf_e, hbuf,
                  h_sh, sem_a, sem_b, sem_e):
    c = lax.axis_index("c")
    s = lax.axis_index("s")
    wid = c * NS + s
    node_base = s * NPT
    out_base = c * NPAD + node_base

    def zero_h(k, _):
        pltpu.sync_copy(zh_hbm, h_sh.at[pl.ds(node_base + k * B, B)])
        return 0
    lax.fori_loop(0, WB, zero_h, 0)
    pltpu.sync_copy(zh_hbm.at[pl.ds(0, WTAIL)],
                    h_sh.at[pl.ds(node_base + WB * B, WTAIL)])

    plsc.subcore_barrier()

    edge_base = wid * EPW

    def chunk(i, _):
        base = edge_base + i * B
        pltpu.sync_copy(row_hbm.at[pl.ds(base, B)], row_v)
        pltpu.sync_copy(col_hbm.at[pl.ds(base, B)], col_v)
        cp_a = pltpu.async_copy(xa_hbm.at[row_v], buf_a, sem_a)
        cp_b = pltpu.async_copy(xb_hbm.at[col_v], buf_b, sem_b)
        cp_e = pltpu.async_copy(ea_hbm.at[pl.ds(base, B)], buf_e, sem_e)
        cp_a.wait()
        cp_b.wait()
        cp_e.wait()

        def edge(e, _):
            for j in range(D // 16):
                sl = pl.ds(j * 16, 16)
                v = buf_a[e, sl] + buf_b[e, sl] + buf_e[e, sl]
                hbuf[e, sl] = jnp.maximum(v, 0.0)
            return 0
        lax.fori_loop(0, B, edge, 0)

        pltpu.sync_copy(hbuf, h_sh.at[col_v], add=True)
        return 0
    lax.fori_loop(0, NCHUNK, chunk, 0)

    plsc.subcore_barrier()

    def wb(k, _):
        off = k * B
        pltpu.sync_copy(h_sh.at[pl.ds(node_base + off, B)], hbuf)
        pltpu.sync_copy(hbuf, h_out.at[pl.ds(out_base + off, B)])
        return 0
    lax.fori_loop(0, WB, wb, 0)
    off2 = WB * B
    pltpu.sync_copy(h_sh.at[pl.ds(node_base + off2, WTAIL)],
                    hbuf.at[pl.ds(0, WTAIL)])
    pltpu.sync_copy(hbuf.at[pl.ds(0, WTAIL)],
                    h_out.at[pl.ds(out_base + off2, WTAIL)])


@functools.partial(
    pl.kernel,
    out_type=jax.ShapeDtypeStruct((NC * NPAD, D), jnp.float32),
    mesh=plsc.VectorSubcoreMesh(core_axis_name="c", subcore_axis_name="s"),
    scratch_types=[
        pltpu.VMEM((B,), jnp.int32),
        pltpu.VMEM((B,), jnp.int32),
        pltpu.VMEM((B, D), jnp.float32),
        pltpu.VMEM((B, D), jnp.float32),
        pltpu.VMEM((B, D), jnp.float32),
        pltpu.VMEM((B, D), jnp.float32),
        pltpu.VMEM_SHARED((NPAD, D), jnp.float32),
        pltpu.SemaphoreType.DMA,
        pltpu.SemaphoreType.DMA,
        pltpu.SemaphoreType.DMA,
    ],
)
def _sc_edge_kernel(*refs):
    _sc_edge_body(*refs)


def kernel(x, edge_index, edge_attr, W1, b1, W2, b2):
    row = edge_index[0]
    col = edge_index[1]
    w1a = W1[:D]
    w1b = W1[D:2 * D]
    w1e = W1[2 * D:]
    b1r = b1.reshape(1, D)
    b2r = b2.reshape(1, D)

    nb = 400
    xa, xb = pl.pallas_call(
        _node_mm_body,
        grid=(N // nb,),
        in_specs=[
            pl.BlockSpec((nb, D), lambda i: (i, 0)),
            pl.BlockSpec((D, D), lambda i: (0, 0)),
            pl.BlockSpec((D, D), lambda i: (0, 0)),
        ],
        out_specs=[
            pl.BlockSpec((nb, D), lambda i: (i, 0)),
            pl.BlockSpec((nb, D), lambda i: (i, 0)),
        ],
        out_shape=[
            jax.ShapeDtypeStruct((N, D), jnp.float32),
            jax.ShapeDtypeStruct((N, D), jnp.float32),
        ],
    )(x, w1a, w1b)

    eb = 1280
    ea = pl.pallas_call(
        _edge_mm_body,
        grid=(E // eb,),
        in_specs=[
            pl.BlockSpec((eb, D_EDGE), lambda i: (i, 0)),
            pl.BlockSpec((D_EDGE, D), lambda i: (0, 0)),
            pl.BlockSpec((1, D), lambda i: (0, 0)),
        ],
        out_specs=pl.BlockSpec((eb, D), lambda i: (i, 0)),
        out_shape=jax.ShapeDtypeStruct((E, D), jnp.float32),
    )(edge_attr, w1e, b1r)

    zh = jnp.zeros((B, D), jnp.float32)
    h_part = _sc_edge_kernel(row, col, ea, xa, xb, zh)

    cb = 3200
    cnt_mat = pl.pallas_call(
        _cnt_mm_body,
        grid=(E // cb,),
        in_specs=[pl.BlockSpec((cb, 1), lambda i: (i, 0))],
        out_specs=pl.BlockSpec((80, D), lambda i: (0, 0)),
        out_shape=jax.ShapeDtypeStruct((80, D), jnp.float32),
    )(col.reshape(E, 1))
    cnt_col = cnt_mat.reshape(80 * D, 1)

    out = pl.pallas_call(
        _final_body,
        grid=(N // nb,),
        in_specs=[
            pl.BlockSpec((nb, D), lambda i: (i, 0)),
            pl.BlockSpec((nb, D), lambda i: (i, 0)),
            pl.BlockSpec((nb, 1), lambda i: (i, 0)),
            pl.BlockSpec((D, D), lambda i: (0, 0)),
            pl.BlockSpec((1, D), lambda i: (0, 0)),
        ],
        out_specs=pl.BlockSpec((nb, D), lambda i: (i, 0)),
        out_shape=jax.ShapeDtypeStruct((N, D), jnp.float32),
    )(h_part[:NPAD], h_part[NPAD:], cnt_col, W2, b2r)

    return out


# speedup vs baseline: 3.1263x; 1.2224x over previous
import functools

import jax
import jax.numpy as jnp
from jax import lax
from jax.experimental import pallas as pl
from jax.experimental.pallas import tpu as pltpu
from jax.experimental.pallas import tpu_sc as plsc

N = 10000
E = 320000
D = 128
DH = 144
D_EDGE = 16

NC = 2
NS = 16
NW = NC * NS
EPW = E // NW
B = 80
NCHUNK = EPW // B
NPAD = 10112
NPT = NPAD // NS
WB = NPT // B
WTAIL = NPT - WB * B


def _node_mm_body(x_ref, wa_ref, wb_ref, xa_ref, xb_ref):
    xv = x_ref[...]
    xa_ref[...] = jnp.dot(xv, wa_ref[...], preferred_element_type=jnp.float32)
    xb_ref[...] = jnp.dot(xv, wb_ref[...], preferred_element_type=jnp.float32)


def _edge_mm_body(attr_ref, we_ref, b1_ref, ea_ref):
    ea_ref[...] = (
        jnp.dot(attr_ref[...], we_ref[...], preferred_element_type=jnp.float32)
        + b1_ref[...]
    )


def _cnt_mm_body(col_ref, cnt_ref):
    i = pl.program_id(0)
    colv = col_ref[...]
    hi = lax.shift_right_logical(colv, 7)
    lo = lax.bitwise_and(colv, 127)
    r80 = lax.broadcasted_iota(jnp.int32, (1, 80), 1)
    l128 = lax.broadcasted_iota(jnp.int32, (1, D), 1)
    m = (hi == r80).astype(jnp.float32)
    p = (lo == l128).astype(jnp.float32)
    acc = lax.dot_general(m, p, (((0,), (0,)), ((), ())),
                          preferred_element_type=jnp.float32)

    @pl.when(i == 0)
    def _():
        cnt_ref[...] = jnp.zeros_like(cnt_ref)

    cnt_ref[...] += acc


def _final_body(h0_ref, h1_ref, cnt_ref, w2_ref, b2_ref, out_ref):
    hsum = h0_ref[...] + h1_ref[...]
    cnt = cnt_ref[...]
    acc = jnp.dot(hsum, w2_ref[...], preferred_element_type=jnp.float32)
    out_ref[...] = (acc + cnt * b2_ref[...]) / jnp.maximum(cnt, 1.0)


def _sc_edge_body(row_hbm, col_hbm, ea_hbm, xa_hbm, xb_hbm, zh_hbm,
                  h_out,
                  row_v, col_v, buf_a, buf_b, buf_e,
                  h_sh, sem_a, sem_b, sem_e):
    c = lax.axis_index("c")
    s = lax.axis_index("s")
    wid = c * NS + s
    node_base = s * NPT
    out_base = c * NPAD + node_base

    def zero_h(k, _):
        pltpu.sync_copy(zh_hbm, h_sh.at[pl.ds(node_base + k * B, B)])
        return 0
    lax.fori_loop(0, WB, zero_h, 0)
    pltpu.sync_copy(zh_hbm.at[pl.ds(0, WTAIL)],
                    h_sh.at[pl.ds(node_base + WB * B, WTAIL)])

    plsc.subcore_barrier()

    edge_base = wid * EPW

    def chunk(i, _):
        base = edge_base + i * B
        pltpu.sync_copy(row_hbm.at[pl.ds(base, B)], row_v)
        pltpu.sync_copy(col_hbm.at[pl.ds(base, B)], col_v)
        cp_a = pltpu.async_copy(xa_hbm.at[row_v], buf_a, sem_a)
        cp_b = pltpu.async_copy(xb_hbm.at[col_v], buf_b, sem_b)
        cp_e = pltpu.async_copy(ea_hbm.at[pl.ds(base, B)], buf_e, sem_e)
        cp_a.wait()
        cp_b.wait()
        cp_e.wait()

        def edge(e, _):
            for j in range(D // 16):
                sl = pl.ds(j * 16, 16)
                v = buf_a[e, sl] + buf_b[e, sl] + buf_e[e, sl]
                buf_e[e, sl] = jnp.maximum(v, 0.0)
            return 0
        lax.fori_loop(0, B, edge, 0)

        pltpu.sync_copy(buf_e, h_sh.at[col_v], add=True)
        return 0
    lax.fori_loop(0, NCHUNK, chunk, 0)

    plsc.subcore_barrier()

    def wb(k, _):
        off = k * B
        pltpu.sync_copy(h_sh.at[pl.ds(node_base + off, B)], buf_a)
        pltpu.sync_copy(buf_a, h_out.at[pl.ds(out_base + off, B)])
        return 0
    lax.fori_loop(0, WB, wb, 0)
    off2 = WB * B
    pltpu.sync_copy(h_sh.at[pl.ds(node_base + off2, WTAIL)],
                    buf_a.at[pl.ds(0, WTAIL)])
    pltpu.sync_copy(buf_a.at[pl.ds(0, WTAIL)],
                    h_out.at[pl.ds(out_base + off2, WTAIL)])


@functools.partial(
    pl.kernel,
    out_type=jax.ShapeDtypeStruct((NC * NPAD, D), jnp.float32),
    mesh=plsc.VectorSubcoreMesh(core_axis_name="c", subcore_axis_name="s"),
    scratch_types=[
        pltpu.VMEM((B,), jnp.int32),
        pltpu.VMEM((B,), jnp.int32),
        pltpu.VMEM((B, D), jnp.float32),
        pltpu.VMEM((B, D), jnp.float32),
        pltpu.VMEM((B, D), jnp.float32),
        pltpu.VMEM_SHARED((NPAD, D), jnp.float32),
        pltpu.SemaphoreType.DMA,
        pltpu.SemaphoreType.DMA,
        pltpu.SemaphoreType.DMA,
    ],
)
def _sc_edge_kernel(*refs):
    _sc_edge_body(*refs)


def kernel(x, edge_index, edge_attr, W1, b1, W2, b2):
    row = edge_index[0]
    col = edge_index[1]
    w1a = W1[:D]
    w1b = W1[D:2 * D]
    w1e = W1[2 * D:]
    b1r = b1.reshape(1, D)
    b2r = b2.reshape(1, D)

    nb = 400
    xa, xb = pl.pallas_call(
        _node_mm_body,
        grid=(N // nb,),
        in_specs=[
            pl.BlockSpec((nb, D), lambda i: (i, 0)),
            pl.BlockSpec((D, D), lambda i: (0, 0)),
            pl.BlockSpec((D, D), lambda i: (0, 0)),
        ],
        out_specs=[
            pl.BlockSpec((nb, D), lambda i: (i, 0)),
            pl.BlockSpec((nb, D), lambda i: (i, 0)),
        ],
        out_shape=[
            jax.ShapeDtypeStruct((N, D), jnp.float32),
            jax.ShapeDtypeStruct((N, D), jnp.float32),
        ],
    )(x, w1a, w1b)

    eb = 1280
    ea = pl.pallas_call(
        _edge_mm_body,
        grid=(E // eb,),
        in_specs=[
            pl.BlockSpec((eb, D_EDGE), lambda i: (i, 0)),
            pl.BlockSpec((D_EDGE, D), lambda i: (0, 0)),
            pl.BlockSpec((1, D), lambda i: (0, 0)),
        ],
        out_specs=pl.BlockSpec((eb, D), lambda i: (i, 0)),
        out_shape=jax.ShapeDtypeStruct((E, D), jnp.float32),
    )(edge_attr, w1e, b1r)

    zh = jnp.zeros((B, D), jnp.float32)
    h_part = _sc_edge_kernel(row, col, ea, xa, xb, zh)

    cb = 3200
    cnt_mat = pl.pallas_call(
        _cnt_mm_body,
        grid=(E // cb,),
        in_specs=[pl.BlockSpec((cb, 1), lambda i: (i, 0))],
        out_specs=pl.BlockSpec((80, D), lambda i: (0, 0)),
        out_shape=jax.ShapeDtypeStruct((80, D), jnp.float32),
    )(col.reshape(E, 1))
    cnt_col = cnt_mat.reshape(80 * D, 1)

    out = pl.pallas_call(
        _final_body,
        grid=(N // nb,),
        in_specs=[
            pl.BlockSpec((nb, D), lambda i: (i, 0)),
            pl.BlockSpec((nb, D), lambda i: (i, 0)),
            pl.BlockSpec((nb, 1), lambda i: (i, 0)),
            pl.BlockSpec((D, D), lambda i: (0, 0)),
            pl.BlockSpec((1, D), lambda i: (0, 0)),
        ],
        out_specs=pl.BlockSpec((nb, D), lambda i: (i, 0)),
        out_shape=jax.ShapeDtypeStruct((N, D), jnp.float32),
    )(h_part[:NPAD], h_part[NPAD:], cnt_col, W2, b2r)

    return out


# speedup vs baseline: 3.3570x; 1.0738x over previous
import functools

import jax
import jax.numpy as jnp
from jax import lax
from jax.experimental import pallas as pl
from jax.experimental.pallas import tpu as pltpu
from jax.experimental.pallas import tpu_sc as plsc

N = 10000
E = 320000
D = 128
DH = 144
D_EDGE = 16

NC = 2
NS = 16
NW = NC * NS
EPW = E // NW
B = 80
NCHUNK = EPW // B
NPAD = 10112
NPT = NPAD // NS
WB = NPT // B
WTAIL = NPT - WB * B


def _node_mm_body(x_ref, wa_ref, wb_ref, xa_ref, xb_ref):
    xv = x_ref[...]
    xa_ref[...] = jnp.dot(xv, wa_ref[...], preferred_element_type=jnp.float32)
    xb_ref[...] = jnp.dot(xv, wb_ref[...], preferred_element_type=jnp.float32)


def _edge_mm_body(attr_ref, we_ref, b1_ref, ea_ref):
    ea_ref[...] = (
        jnp.dot(attr_ref[...], we_ref[...], preferred_element_type=jnp.float32)
        + b1_ref[...]
    )


def _cnt_mm_body(col_ref, cnt_ref):
    i = pl.program_id(0)
    colv = col_ref[...]
    hi = lax.shift_right_logical(colv, 7)
    lo = lax.bitwise_and(colv, 127)
    r80 = lax.broadcasted_iota(jnp.int32, (1, 80), 1)
    l128 = lax.broadcasted_iota(jnp.int32, (1, D), 1)
    m = (hi == r80).astype(jnp.float32)
    p = (lo == l128).astype(jnp.float32)
    acc = lax.dot_general(m, p, (((0,), (0,)), ((), ())),
                          preferred_element_type=jnp.float32)

    @pl.when(i == 0)
    def _():
        cnt_ref[...] = jnp.zeros_like(cnt_ref)

    cnt_ref[...] += acc


def _final_body(h0_ref, h1_ref, cnt_ref, w2_ref, b2_ref, out_ref):
    hsum = h0_ref[...] + h1_ref[...]
    cnt = cnt_ref[...]
    acc = jnp.dot(hsum, w2_ref[...], preferred_element_type=jnp.float32)
    out_ref[...] = (acc + cnt * b2_ref[...]) / jnp.maximum(cnt, 1.0)


def _sc_edge_body(pk_hbm, ea_hbm, xa_hbm, xb_hbm, zh_hbm,
                  h_out,
                  pk_v, row_v, col_v, buf_a, buf_b, buf_e,
                  h_sh, sem_a, sem_b, sem_e):
    c = lax.axis_index("c")
    s = lax.axis_index("s")
    wid = c * NS + s
    node_base = s * NPT
    out_base = c * NPAD + node_base

    def zero_h(k, _):
        pltpu.sync_copy(zh_hbm, h_sh.at[pl.ds(node_base + k * B, B)])
        return 0
    lax.fori_loop(0, WB, zero_h, 0)
    pltpu.sync_copy(zh_hbm.at[pl.ds(0, WTAIL)],
                    h_sh.at[pl.ds(node_base + WB * B, WTAIL)])

    plsc.subcore_barrier()

    pltpu.sync_copy(pk_hbm.at[pl.ds(wid * EPW, EPW)], pk_v)

    def chunk(i, _):
        base = i * B
        for g in range(B // 16):
            w = pk_v[pl.ds(base + g * 16, 16)]
            row_v[pl.ds(g * 16, 16)] = lax.shift_right_logical(w, 14)
            col_v[pl.ds(g * 16, 16)] = lax.bitwise_and(w, 16383)
        cp_a = pltpu.async_copy(xa_hbm.at[row_v], buf_a, sem_a)
        cp_b = pltpu.async_copy(xb_hbm.at[col_v], buf_b, sem_b)
        cp_e = pltpu.async_copy(ea_hbm.at[pl.ds(wid * EPW + base, B)], buf_e, sem_e)
        cp_a.wait()
        cp_b.wait()
        cp_e.wait()

        def edge(e, _):
            for j in range(D // 16):
                sl = pl.ds(j * 16, 16)
                v = buf_a[e, sl] + buf_b[e, sl] + buf_e[e, sl]
                buf_e[e, sl] = jnp.maximum(v, 0.0)
            return 0
        lax.fori_loop(0, B, edge, 0)

        pltpu.sync_copy(buf_e, h_sh.at[col_v], add=True)
        return 0
    lax.fori_loop(0, NCHUNK, chunk, 0)

    plsc.subcore_barrier()

    def wb(k, _):
        off = k * B
        pltpu.sync_copy(h_sh.at[pl.ds(node_base + off, B)], buf_a)
        pltpu.sync_copy(buf_a, h_out.at[pl.ds(out_base + off, B)])
        return 0
    lax.fori_loop(0, WB, wb, 0)
    off2 = WB * B
    pltpu.sync_copy(h_sh.at[pl.ds(node_base + off2, WTAIL)],
                    buf_a.at[pl.ds(0, WTAIL)])
    pltpu.sync_copy(buf_a.at[pl.ds(0, WTAIL)],
                    h_out.at[pl.ds(out_base + off2, WTAIL)])


@functools.partial(
    pl.kernel,
    out_type=jax.ShapeDtypeStruct((NC * NPAD, D), jnp.float32),
    mesh=plsc.VectorSubcoreMesh(core_axis_name="c", subcore_axis_name="s"),
    scratch_types=[
        pltpu.VMEM((EPW,), jnp.int32),
        pltpu.VMEM((B,), jnp.int32),
        pltpu.VMEM((B,), jnp.int32),
        pltpu.VMEM((B, D), jnp.float32),
        pltpu.VMEM((B, D), jnp.float32),
        pltpu.VMEM((B, D), jnp.float32),
        pltpu.VMEM_SHARED((NPAD, D), jnp.float32),
        pltpu.SemaphoreType.DMA,
        pltpu.SemaphoreType.DMA,
        pltpu.SemaphoreType.DMA,
    ],
)
def _sc_edge_kernel(*refs):
    _sc_edge_body(*refs)


def kernel(x, edge_index, edge_attr, W1, b1, W2, b2):
    row = edge_index[0]
    col = edge_index[1]
    w1a = W1[:D]
    w1b = W1[D:2 * D]
    w1e = W1[2 * D:]
    b1r = b1.reshape(1, D)
    b2r = b2.reshape(1, D)

    nb = 400
    xa, xb = pl.pallas_call(
        _node_mm_body,
        grid=(N // nb,),
        in_specs=[
            pl.BlockSpec((nb, D), lambda i: (i, 0)),
            pl.BlockSpec((D, D), lambda i: (0, 0)),
            pl.BlockSpec((D, D), lambda i: (0, 0)),
        ],
        out_specs=[
            pl.BlockSpec((nb, D), lambda i: (i, 0)),
            pl.BlockSpec((nb, D), lambda i: (i, 0)),
        ],
        out_shape=[
            jax.ShapeDtypeStruct((N, D), jnp.float32),
            jax.ShapeDtypeStruct((N, D), jnp.float32),
        ],
    )(x, w1a, w1b)

    eb = 1280
    ea = pl.pallas_call(
        _edge_mm_body,
        grid=(E // eb,),
        in_specs=[
            pl.BlockSpec((eb, D_EDGE), lambda i: (i, 0)),
            pl.BlockSpec((D_EDGE, D), lambda i: (0, 0)),
            pl.BlockSpec((1, D), lambda i: (0, 0)),
        ],
        out_specs=pl.BlockSpec((eb, D), lambda i: (i, 0)),
        out_shape=jax.ShapeDtypeStruct((E, D), jnp.float32),
    )(edge_attr, w1e, b1r)

    zh = jnp.zeros((B, D), jnp.float32)
    packed = jnp.bitwise_or(jnp.left_shift(row, 14), col)
    h_part = _sc_edge_kernel(packed, ea, xa, xb, zh)

    cb = 3200
    cnt_mat = pl.pallas_call(
        _cnt_mm_body,
        grid=(E // cb,),
        in_specs=[pl.BlockSpec((cb, 1), lambda i: (i, 0))],
        out_specs=pl.BlockSpec((80, D), lambda i: (0, 0)),
        out_shape=jax.ShapeDtypeStruct((80, D), jnp.float32),
    )(col.reshape(E, 1))
    cnt_col = cnt_mat.reshape(80 * D, 1)

    out = pl.pallas_call(
        _final_body,
        grid=(N // nb,),
        in_specs=[
            pl.BlockSpec((nb, D), lambda i: (i, 0)),
            pl.BlockSpec((nb, D), lambda i: (i, 0)),
            pl.BlockSpec((nb, 1), lambda i: (i, 0)),
            pl.BlockSpec((D, D), lambda i: (0, 0)),
            pl.BlockSpec((1, D), lambda i: (0, 0)),
        ],
        out_specs=pl.BlockSpec((nb, D), lambda i: (i, 0)),
        out_shape=jax.ShapeDtypeStruct((N, D), jnp.float32),
    )(h_part[:NPAD], h_part[NPAD:], cnt_col, W2, b2r)

    return out
